# baseline (speedup 1.0000x reference)
import jax
import jax.numpy as jnp
from jax.experimental import pallas as pl

VOCAB_DEPTH = 1050
ROWS = 1024 * 20
BLOCK_ROWS = 256


def _onehot_body(tok_ref, out_ref):
    tok = tok_ref[...]
    cols = jax.lax.broadcasted_iota(jnp.int32, (BLOCK_ROWS, VOCAB_DEPTH), 1)
    out_ref[...] = (cols == tok).astype(jnp.float32)


def kernel(tokens):
    b, s = tokens.shape
    tok_flat = tokens.reshape(b * s, 1).astype(jnp.int32)
    out = pl.pallas_call(
        _onehot_body,
        grid=(ROWS // BLOCK_ROWS,),
        in_specs=[pl.BlockSpec((BLOCK_ROWS, 1), lambda i: (i, 0))],
        out_specs=pl.BlockSpec((BLOCK_ROWS, VOCAB_DEPTH), lambda i: (i, 0)),
        out_shape=jax.ShapeDtypeStruct((ROWS, VOCAB_DEPTH), jnp.float32),
    )(tok_flat)
    return out.reshape(b, s, VOCAB_DEPTH)


# speedup vs baseline: 8.1716x; 8.1716x over previous
import jax
import jax.numpy as jnp
from jax.experimental import pallas as pl

VOCAB_DEPTH = 1050
SEQ = 20
BATCH = 1024
BATCH_BLOCK = 256


def _onehot_body(tok_ref, out_ref):
    tok = tok_ref[...]
    d = jax.lax.broadcasted_iota(jnp.int32, (SEQ, VOCAB_DEPTH, BATCH_BLOCK), 1)
    out_ref[...] = (d == tok[:, None, :]).astype(jnp.float32)


def kernel(tokens):
    b, s = tokens.shape
    tok_t = tokens.T.astype(jnp.int32)
    out = pl.pallas_call(
        _onehot_body,
        grid=(b // BATCH_BLOCK,),
        in_specs=[pl.BlockSpec((s, BATCH_BLOCK), lambda i: (0, i))],
        out_specs=pl.BlockSpec((s, VOCAB_DEPTH, BATCH_BLOCK), lambda i: (0, 0, i)),
        out_shape=jax.ShapeDtypeStruct((s, VOCAB_DEPTH, b), jnp.float32),
    )(tok_t)
    return out.transpose(2, 0, 1)
